# baseline (speedup 1.0000x reference)
import functools

import jax
import jax.numpy as jnp
from jax import lax
from jax.experimental import pallas as pl
from jax.experimental.pallas import tpu as pltpu
from jax.experimental.pallas import tpu_sc as plsc

DIM = 64
BATCH = 16384
HALF = BATCH // 2
MARGIN = 1.0

NW = 32
BPW = BATCH // NW
CHUNK = 128
NCHUNK = BPW // CHUNK
NBUF = 2
L = 16


def _sc_scores(h_idx, t_idx, r_idx, ent_re, ent_im, rel_re, rel_im):
    mesh = plsc.VectorSubcoreMesh(core_axis_name="c", subcore_axis_name="s")
    row_buf = lambda: pltpu.VMEM((CHUNK, DIM), jnp.float32)

    @functools.partial(
        pl.kernel,
        mesh=mesh,
        compiler_params=pltpu.CompilerParams(
            needs_layout_passes=False, use_tc_tiling_on_sc=False
        ),
        out_type=jax.ShapeDtypeStruct((BATCH,), jnp.float32),
        scratch_types=(
            [pltpu.VMEM((BPW,), jnp.int32) for _ in range(3)]
            + [row_buf() for _ in range(6 * NBUF)]
            + [pltpu.VMEM((BPW,), jnp.float32)]
            + [pltpu.SemaphoreType.DMA for _ in range(NBUF)]
        ),
    )
    def k(h_hbm, t_hbm, r_hbm, ere_hbm, eim_hbm, rre_hbm, rim_hbm, out_hbm,
          ih, it, ir, *rest):
        bufs = [rest[6 * s:6 * (s + 1)] for s in range(NBUF)]
        score = rest[6 * NBUF]
        sems = rest[6 * NBUF + 1:]
        wid = lax.axis_index("s") * 2 + lax.axis_index("c")
        base = wid * BPW
        pltpu.sync_copy(h_hbm.at[pl.ds(base, BPW)], ih)
        pltpu.sync_copy(t_hbm.at[pl.ds(base, BPW)], it)
        pltpu.sync_copy(r_hbm.at[pl.ds(base, BPW)], ir)

        def start(c):
            s = c % NBUF
            hre, him, tre, tim, rre, rim = bufs[s]
            hh = ih.at[pl.ds(c * CHUNK, CHUNK)]
            tt = it.at[pl.ds(c * CHUNK, CHUNK)]
            rr = ir.at[pl.ds(c * CHUNK, CHUNK)]
            return [
                pltpu.async_copy(ere_hbm.at[hh], hre, sems[s]),
                pltpu.async_copy(eim_hbm.at[hh], him, sems[s]),
                pltpu.async_copy(ere_hbm.at[tt], tre, sems[s]),
                pltpu.async_copy(eim_hbm.at[tt], tim, sems[s]),
                pltpu.async_copy(rre_hbm.at[rr], rre, sems[s]),
                pltpu.async_copy(rim_hbm.at[rr], rim, sems[s]),
            ]

        iota = lax.iota(jnp.int32, L)

        def compute(c):
            hre, him, tre, tim, rre, rim = bufs[c % NBUF]
            for g in range(CHUNK // L):
                rows = iota + (g * L)

                def body(d, acc):
                    cols = lax.broadcast(d, (L,))
                    xhre = plsc.load_gather(hre, [rows, cols])
                    xhim = plsc.load_gather(him, [rows, cols])
                    xtre = plsc.load_gather(tre, [rows, cols])
                    xtim = plsc.load_gather(tim, [rows, cols])
                    xrre = plsc.load_gather(rre, [rows, cols])
                    xrim = plsc.load_gather(rim, [rows, cols])
                    re_part = xhre * xtre + xhim * xtim
                    im_part = xhre * xtim - xhim * xtre
                    return acc + (xrre * re_part + xrim * im_part)

                acc = lax.fori_loop(0, DIM, body, jnp.zeros((L,), jnp.float32))
                score[pl.ds(c * CHUNK + g * L, L)] = -acc

        handles = start(0)
        for c in range(NCHUNK):
            nxt = start(c + 1) if c + 1 < NCHUNK else None
            for hnd in handles:
                hnd.wait()
            compute(c)
            handles = nxt
        pltpu.sync_copy(score, out_hbm.at[pl.ds(base, BPW)])

    return k(h_idx, t_idx, r_idx, ent_re, ent_im, rel_re, rel_im)


def _loss_body(s_ref, out_ref):
    s = s_ref[...]
    pos = s[:HALF // 128, :]
    neg = s[HALF // 128:, :]
    out_ref[0, 0] = jnp.sum(jnp.maximum(pos - neg + MARGIN, 0.0))


_tc_loss = pl.pallas_call(
    _loss_body,
    out_shape=jax.ShapeDtypeStruct((1, 1), jnp.float32),
    out_specs=pl.BlockSpec(memory_space=pltpu.SMEM),
)


def kernel(batch_h, batch_t, batch_r, batch_y, ent_re, ent_im, rel_re, rel_im):
    del batch_y
    h = batch_h.astype(jnp.int32)
    t = batch_t.astype(jnp.int32)
    r = batch_r.astype(jnp.int32)
    score = _sc_scores(h, t, r, ent_re, ent_im, rel_re, rel_im)
    loss = _tc_loss(score.reshape(BATCH // 128, 128))[0, 0]
    return (loss, score[:HALF], score[HALF:])
